# baseline (speedup 1.0000x reference)
import functools

import jax
import jax.numpy as jnp
from jax import lax
from jax.experimental import pallas as pl
from jax.experimental.pallas import tpu as pltpu
from jax.experimental.pallas import tpu_sc as plsc

NEG_SLOPE = 0.2
L = 16
BW = 64
CH = 32


def _tc1_body(x_ref, w_ref, ap_ref, h_ref, s_ref):
    h = lax.dot_general(x_ref[...], w_ref[...], (((1,), (1,)), ((), ())),
                        preferred_element_type=jnp.float32)
    h_ref[...] = h
    s_ref[...] = lax.dot_general(h, ap_ref[...], (((1,), (0,)), ((), ())),
                                 preferred_element_type=jnp.float32)


def _make_sc_kernel(n, n_pad, rows_sc, e_real):
    mesh = plsc.VectorSubcoreMesh(core_axis_name="c", subcore_axis_name="s",
                                  num_cores=1)
    stripe = n_pad // 16
    n_chunks = rows_sc // CH
    g_all = 4

    rows_all = rows_sc * 16

    @functools.partial(
        pl.kernel,
        out_type=(jax.ShapeDtypeStruct((n_pad, 128), jnp.float32),
                  jax.ShapeDtypeStruct((rows_all, BW), jnp.float32)),
        mesh=mesh,
        scratch_types=dict(
            sdst_v=pltpu.VMEM((n_pad,), jnp.float32),
            ssrc_v=pltpu.VMEM((n_pad,), jnp.float32),
            colc=pltpu.VMEM((CH, BW), jnp.int32),
            rowc=pltpu.VMEM((CH, BW), jnp.int32),
            eexp_v=pltpu.VMEM((CH, BW), jnp.float32),
            rows_v=pltpu.VMEM((2, BW, 128), jnp.float32),
            gsg=pltpu.VMEM((BW,), jnp.float32),
            alpha_v=pltpu.VMEM((BW,), jnp.float32),
            idxb=pltpu.VMEM((2, 128), jnp.int32),
            maxtmp=pltpu.VMEM((L,), jnp.float32),
            maxv=pltpu.VMEM((16, L), jnp.float32),
            shared_out=pltpu.VMEM_SHARED((n_pad, 128), jnp.float32),
            shared_sum=pltpu.VMEM_SHARED((n_pad,), jnp.float32),
            shared_m1=pltpu.VMEM_SHARED((16, L), jnp.float32),
            shared_m2=pltpu.VMEM_SHARED((16, L), jnp.float32),
            sem_g=pltpu.SemaphoreType.DMA,
            sem_s=pltpu.SemaphoreType.DMA,
        ),
        compiler_params=pltpu.CompilerParams(needs_layout_passes=False,
                                             use_tc_tiling_on_sc=False),
    )
    def sc_kernel(row_hbm, col_hbm, sdst_hbm, ssrc_hbm, h_hbm, out_hbm,
                  eexp_hbm, sdst_v, ssrc_v, colc, rowc, eexp_v, rows_v, gsg,
                  alpha_v, idxb, maxtmp, maxv, shared_out, shared_sum,
                  shared_m1, shared_m2, sem_g, sem_s):
        s = lax.axis_index("s")
        iota = lax.iota(jnp.int32, L)

        def zrows(j, _):
            for dlo in range(8):
                rows_v[0, j, pl.ds(dlo * L, L)] = jnp.zeros((L,), jnp.float32)
            return 0

        lax.fori_loop(0, BW, zrows, 0)
        zcopies = []
        for q in range(stripe // 128):
            if len(zcopies) >= 3:
                zcopies.pop(0).wait()
            zcopies.append(pltpu.async_copy(
                rows_v.at[0, 0],
                shared_sum.at[pl.ds(s * stripe + q * 128, 128)], sem_s))
        for q in range(stripe // BW):
            if len(zcopies) >= 3:
                zcopies.pop(0).wait()
            zcopies.append(pltpu.async_copy(
                rows_v.at[0],
                shared_out.at[pl.ds(s * stripe + q * BW, BW)], sem_g))

        pltpu.sync_copy(sdst_hbm, sdst_v)
        pltpu.sync_copy(ssrc_hbm, ssrc_v)

        def tmax_body(q, carry):
            v1, v2 = carry
            off = s * stripe + q * L
            v1 = jnp.maximum(v1, sdst_v[pl.ds(off, L)])
            v2 = jnp.maximum(v2, ssrc_v[pl.ds(off, L)])
            return v1, v2

        neg = jnp.full((L,), -3e38, jnp.float32)
        v1, v2 = lax.fori_loop(0, stripe // L, tmax_body, (neg, neg))
        maxtmp[...] = jnp.full((L,), jnp.max(v1), jnp.float32)
        pltpu.sync_copy(maxtmp, shared_m1.at[s])
        maxtmp[...] = jnp.full((L,), jnp.max(v2), jnp.float32)
        pltpu.sync_copy(maxtmp, shared_m2.at[s])
        for zc in zcopies:
            zc.wait()
        plsc.subcore_barrier()

        pltpu.sync_copy(shared_m1, maxv)
        m1 = maxv[0, :]
        for t in range(1, 16):
            m1 = jnp.maximum(m1, maxv[t, :])
        pltpu.sync_copy(shared_m2, maxv)
        m2 = maxv[0, :]
        for t in range(1, 16):
            m2 = jnp.maximum(m2, maxv[t, :])
        cc = jnp.maximum(jnp.max(m1) + jnp.max(m2), 0.0)

        base_t = s * (rows_sc * BW)
        rows_real = jnp.clip((e_real - base_t + BW - 1) // BW, 0, rows_sc)
        chunks_real = (rows_real + CH - 1) // CH

        def eexp16(jj, i, cq):
            ci = colc[jj, pl.ds(i * L, L)]
            ri = rowc[jj, pl.ds(i * L, L)]
            e = (plsc.load_gather(sdst_v, [ci])
                 + plsc.load_gather(ssrc_v, [ri]))
            e = jnp.where(e >= 0.0, e, NEG_SLOPE * e)
            g = base_t + cq * (CH * BW) + jj * BW + i * L + iota
            e = jnp.where(g < e_real, e, -1e30)
            return jnp.exp(e - cc)

        def p1_body(cq, _):
            pltpu.sync_copy(row_hbm.at[pl.ds(s * rows_sc + cq * CH, CH)],
                            rowc)
            pltpu.sync_copy(col_hbm.at[pl.ds(s * rows_sc + cq * CH, CH)],
                            colc)
            for jj in range(CH):
                for i in range(g_all):
                    eexp_v[jj, pl.ds(i * L, L)] = eexp16(jj, i, cq)
            pltpu.sync_copy(eexp_v,
                            eexp_hbm.at[pl.ds(s * rows_sc + cq * CH, CH)])
            return 0

        lax.fori_loop(0, chunks_real, p1_body, 0)

        def zsum(q, _):
            ssrc_v[pl.ds(q * L, L)] = jnp.zeros((L,), jnp.float32)
            return 0

        lax.fori_loop(0, n_pad // L, zsum, 0)

        def p1b_body(cq, _):
            pltpu.sync_copy(col_hbm.at[pl.ds(s * rows_sc + cq * CH, CH)],
                            colc)
            pltpu.sync_copy(eexp_hbm.at[pl.ds(s * rows_sc + cq * CH, CH)],
                            eexp_v)
            for jj in range(CH):
                for i in range(g_all):
                    ci = colc[jj, pl.ds(i * L, L)]
                    plsc.addupdate_scatter(ssrc_v, [ci],
                                           eexp_v[jj, pl.ds(i * L, L)])
            return 0

        lax.fori_loop(0, chunks_real, p1b_body, 0)

        pend = [None, None]
        for b in range(n_pad // 128):
            buf = b % 2
            if pend[buf] is not None:
                pend[buf].wait()
            for g in range(8):
                idxb[buf, pl.ds(g * L, L)] = b * 128 + g * L + iota
            pend[buf] = pltpu.async_copy(
                ssrc_v.at[pl.ds(b * 128, 128)], shared_sum.at[idxb.at[buf]],
                sem_s, add=True)
        for p in pend:
            if p is not None:
                p.wait()
        plsc.subcore_barrier()

        pltpu.sync_copy(shared_sum, sdst_v)

        def recip_body(q, _):
            v = sdst_v[pl.ds(q * L, L)]
            sdst_v[pl.ds(q * L, L)] = 1.0 / v
            return 0

        lax.fori_loop(0, n_pad // L, recip_body, 0)

        def p2_body(cq, _):
            pltpu.sync_copy(row_hbm.at[pl.ds(s * rows_sc + cq * CH, CH)],
                            rowc)
            pltpu.sync_copy(col_hbm.at[pl.ds(s * rows_sc + cq * CH, CH)],
                            colc)
            pltpu.sync_copy(eexp_hbm.at[pl.ds(s * rows_sc + cq * CH, CH)],
                            eexp_v)
            gets = [None, None]
            puts = [None, None]
            gets[0] = pltpu.async_copy(h_hbm.at[rowc.at[0]], rows_v.at[0],
                                       sem_g)
            for jj in range(CH):
                cur = jj % 2
                nxt = (jj + 1) % 2
                if puts[nxt] is not None:
                    puts[nxt].wait()
                    puts[nxt] = None
                if jj + 1 < CH:
                    gets[nxt] = pltpu.async_copy(
                        h_hbm.at[rowc.at[jj + 1]], rows_v.at[nxt], sem_g)
                for i in range(g_all):
                    ci = colc[jj, pl.ds(i * L, L)]
                    alpha_v[pl.ds(i * L, L)] = (
                        eexp_v[jj, pl.ds(i * L, L)]
                        * plsc.load_gather(sdst_v, [ci]))
                gets[cur].wait()

                def scale(k, _):
                    for u in range(4):
                        kk = k * 4 + u
                        av = plsc.load_gather(
                            alpha_v, [jnp.full((L,), kk, jnp.int32)])
                        for dlo in range(8):
                            rows_v[cur, kk, pl.ds(dlo * L, L)] = (
                                rows_v[cur, kk, pl.ds(dlo * L, L)] * av)
                    return 0

                lax.fori_loop(0, BW // 4, scale, 0)
                puts[cur] = pltpu.async_copy(
                    rows_v.at[cur], shared_out.at[colc.at[jj]], sem_s,
                    add=True)
            for p in puts:
                if p is not None:
                    p.wait()
            return 0

        lax.fori_loop(0, chunks_real, p2_body, 0)
        plsc.subcore_barrier()

        pltpu.sync_copy(shared_out.at[pl.ds(s * stripe, stripe)],
                        out_hbm.at[pl.ds(s * stripe, stripe)])

    return sc_kernel


def kernel(x, edge_index, num_nodes, W, a):
    n, d_in = x.shape
    d_out = W.shape[0]
    e_cnt = edge_index.shape[1]
    lp = jnp.arange(n, dtype=jnp.int32)
    ei = jnp.concatenate(
        [edge_index.astype(jnp.int32), jnp.stack([lp, lp])], axis=1)
    e_total = e_cnt + n
    ep = -(-e_total // (16 * CH * BW)) * (16 * CH * BW)
    ei = jnp.pad(ei, ((0, 0), (0, ep - e_total)))
    rows64 = ep // BW
    row2d = ei[0].reshape(rows64, BW)
    col2d = ei[1].reshape(rows64, BW)

    a1 = a[:d_out, 0]
    a2 = a[d_out:, 0]
    apad = jnp.zeros((d_out, 128), jnp.float32)
    apad = apad.at[:, 0].set(a1).at[:, 1].set(a2)

    blk = 1000
    grid = n // blk
    h, sall = pl.pallas_call(
        _tc1_body,
        grid=(grid,),
        in_specs=[
            pl.BlockSpec((blk, d_in), lambda i: (i, 0)),
            pl.BlockSpec((d_out, d_in), lambda i: (0, 0)),
            pl.BlockSpec((d_out, 128), lambda i: (0, 0)),
        ],
        out_specs=[
            pl.BlockSpec((blk, 128), lambda i: (i, 0)),
            pl.BlockSpec((blk, 128), lambda i: (i, 0)),
        ],
        out_shape=[
            jax.ShapeDtypeStruct((n, 128), jnp.float32),
            jax.ShapeDtypeStruct((n, 128), jnp.float32),
        ],
    )(x, W, apad)

    n_pad = -(-n // (16 * 128)) * (16 * 128)
    pad_n = n_pad - n
    sdst = jnp.pad(sall[:, 0], (0, pad_n), constant_values=-3e38)
    ssrc = jnp.pad(sall[:, 1], (0, pad_n), constant_values=-3e38)

    sc = _make_sc_kernel(n, n_pad, rows64 // 16, e_total)
    partial, _ = sc(row2d, col2d, sdst, ssrc, h)
    return partial[:n]


# speedup vs baseline: 1.0535x; 1.0535x over previous
import functools

import jax
import jax.numpy as jnp
from jax import lax
from jax.experimental import pallas as pl
from jax.experimental.pallas import tpu as pltpu
from jax.experimental.pallas import tpu_sc as plsc

NEG_SLOPE = 0.2
L = 16
BW = 64
CH = 32


def _tc1_body(x_ref, w_ref, ap_ref, h_ref, s_ref):
    h = lax.dot_general(x_ref[...], w_ref[...], (((1,), (1,)), ((), ())),
                        preferred_element_type=jnp.float32)
    h_ref[...] = h
    s_ref[...] = lax.dot_general(h, ap_ref[...], (((1,), (0,)), ((), ())),
                                 preferred_element_type=jnp.float32)


def _make_sc_kernel(n, n_pad, rows_sc, e_real):
    mesh = plsc.VectorSubcoreMesh(core_axis_name="c", subcore_axis_name="s",
                                  num_cores=1)
    stripe = n_pad // 16
    n_chunks = rows_sc // CH
    g_all = 4

    rows_all = rows_sc * 16

    @functools.partial(
        pl.kernel,
        out_type=(jax.ShapeDtypeStruct((n_pad, 128), jnp.float32),
                  jax.ShapeDtypeStruct((rows_all, BW), jnp.float32)),
        mesh=mesh,
        scratch_types=dict(
            sdst_v=pltpu.VMEM((n_pad,), jnp.float32),
            ssrc_v=pltpu.VMEM((n_pad,), jnp.float32),
            colc=pltpu.VMEM((CH, BW), jnp.int32),
            rowc=pltpu.VMEM((CH, BW), jnp.int32),
            eexp_v=pltpu.VMEM((CH, BW), jnp.float32),
            rows_v=pltpu.VMEM((2, BW, 128), jnp.float32),
            gsg=pltpu.VMEM((BW,), jnp.float32),
            alpha_v=pltpu.VMEM((BW,), jnp.float32),
            maxtmp=pltpu.VMEM((L,), jnp.float32),
            maxv=pltpu.VMEM((16, L), jnp.float32),
            shared_out=pltpu.VMEM_SHARED((n_pad, 128), jnp.float32),
            shared_sum=pltpu.VMEM_SHARED((n_pad,), jnp.float32),
            shared_m1=pltpu.VMEM_SHARED((16, L), jnp.float32),
            shared_m2=pltpu.VMEM_SHARED((16, L), jnp.float32),
            sem_g=pltpu.SemaphoreType.DMA,
            sem_s=pltpu.SemaphoreType.DMA,
        ),
        compiler_params=pltpu.CompilerParams(needs_layout_passes=False,
                                             use_tc_tiling_on_sc=False),
    )
    def sc_kernel(row_hbm, col_hbm, sdst_hbm, ssrc_hbm, h_hbm, out_hbm,
                  eexp_hbm, sdst_v, ssrc_v, colc, rowc, eexp_v, rows_v, gsg,
                  alpha_v, maxtmp, maxv, shared_out, shared_sum, shared_m1,
                  shared_m2, sem_g, sem_s):
        s = lax.axis_index("s")
        iota = lax.iota(jnp.int32, L)

        def zrows(j, _):
            for dlo in range(8):
                rows_v[0, j, pl.ds(dlo * L, L)] = jnp.zeros((L,), jnp.float32)
            return 0

        lax.fori_loop(0, BW, zrows, 0)
        zcopies = []
        for q in range(stripe // 128):
            if len(zcopies) >= 3:
                zcopies.pop(0).wait()
            zcopies.append(pltpu.async_copy(
                rows_v.at[0, 0],
                shared_sum.at[pl.ds(s * stripe + q * 128, 128)], sem_s))
        for q in range(stripe // BW):
            if len(zcopies) >= 3:
                zcopies.pop(0).wait()
            zcopies.append(pltpu.async_copy(
                rows_v.at[0],
                shared_out.at[pl.ds(s * stripe + q * BW, BW)], sem_g))

        pltpu.sync_copy(sdst_hbm, sdst_v)
        pltpu.sync_copy(ssrc_hbm, ssrc_v)

        def tmax_body(q, carry):
            v1, v2 = carry
            off = s * stripe + q * L
            v1 = jnp.maximum(v1, sdst_v[pl.ds(off, L)])
            v2 = jnp.maximum(v2, ssrc_v[pl.ds(off, L)])
            return v1, v2

        neg = jnp.full((L,), -3e38, jnp.float32)
        v1, v2 = lax.fori_loop(0, stripe // L, tmax_body, (neg, neg))
        maxtmp[...] = jnp.full((L,), jnp.max(v1), jnp.float32)
        pltpu.sync_copy(maxtmp, shared_m1.at[s])
        maxtmp[...] = jnp.full((L,), jnp.max(v2), jnp.float32)
        pltpu.sync_copy(maxtmp, shared_m2.at[s])
        for zc in zcopies:
            zc.wait()
        plsc.subcore_barrier()

        pltpu.sync_copy(shared_m1, maxv)
        m1 = maxv[0, :]
        for t in range(1, 16):
            m1 = jnp.maximum(m1, maxv[t, :])
        pltpu.sync_copy(shared_m2, maxv)
        m2 = maxv[0, :]
        for t in range(1, 16):
            m2 = jnp.maximum(m2, maxv[t, :])
        cc = jnp.maximum(jnp.max(m1) + jnp.max(m2), 0.0)

        base_t = s * (rows_sc * BW)
        rows_real = jnp.clip((e_real - base_t + BW - 1) // BW, 0, rows_sc)
        chunks_real = (rows_real + CH - 1) // CH

        def eexp16(jj, i, cq):
            ci = colc[jj, pl.ds(i * L, L)]
            ri = rowc[jj, pl.ds(i * L, L)]
            e = (plsc.load_gather(sdst_v, [ci])
                 + plsc.load_gather(ssrc_v, [ri]))
            e = jnp.where(e >= 0.0, e, NEG_SLOPE * e)
            g = base_t + cq * (CH * BW) + jj * BW + i * L + iota
            e = jnp.where(g < e_real, e, -1e30)
            return jnp.exp(e - cc)

        def p1_body(cq, _):
            pltpu.sync_copy(row_hbm.at[pl.ds(s * rows_sc + cq * CH, CH)],
                            rowc)
            pltpu.sync_copy(col_hbm.at[pl.ds(s * rows_sc + cq * CH, CH)],
                            colc)
            puts = [None, None]
            for jj in range(CH):
                for i in range(g_all):
                    eexp_v[jj, pl.ds(i * L, L)] = eexp16(jj, i, cq)
                if puts[jj % 2] is not None:
                    puts[jj % 2].wait()
                puts[jj % 2] = pltpu.async_copy(
                    eexp_v.at[jj], shared_sum.at[colc.at[jj]], sem_s,
                    add=True)
            for p in puts:
                if p is not None:
                    p.wait()
            pltpu.sync_copy(eexp_v,
                            eexp_hbm.at[pl.ds(s * rows_sc + cq * CH, CH)])
            return 0

        lax.fori_loop(0, chunks_real, p1_body, 0)
        plsc.subcore_barrier()

        pltpu.sync_copy(shared_sum, sdst_v)

        def recip_body(q, _):
            v = sdst_v[pl.ds(q * L, L)]
            sdst_v[pl.ds(q * L, L)] = 1.0 / v
            return 0

        lax.fori_loop(0, n_pad // L, recip_body, 0)

        def p2_body(cq, _):
            pltpu.sync_copy(row_hbm.at[pl.ds(s * rows_sc + cq * CH, CH)],
                            rowc)
            pltpu.sync_copy(col_hbm.at[pl.ds(s * rows_sc + cq * CH, CH)],
                            colc)
            pltpu.sync_copy(eexp_hbm.at[pl.ds(s * rows_sc + cq * CH, CH)],
                            eexp_v)
            gets = [None, None]
            puts = [None, None]
            gets[0] = pltpu.async_copy(h_hbm.at[rowc.at[0]], rows_v.at[0],
                                       sem_g)
            for jj in range(CH):
                cur = jj % 2
                nxt = (jj + 1) % 2
                if puts[nxt] is not None:
                    puts[nxt].wait()
                    puts[nxt] = None
                if jj + 1 < CH:
                    gets[nxt] = pltpu.async_copy(
                        h_hbm.at[rowc.at[jj + 1]], rows_v.at[nxt], sem_g)
                for i in range(g_all):
                    ci = colc[jj, pl.ds(i * L, L)]
                    alpha_v[pl.ds(i * L, L)] = (
                        eexp_v[jj, pl.ds(i * L, L)]
                        * plsc.load_gather(sdst_v, [ci]))
                gets[cur].wait()

                def scale(k, _):
                    for u in range(4):
                        kk = k * 4 + u
                        av = plsc.load_gather(
                            alpha_v, [jnp.full((L,), kk, jnp.int32)])
                        for dlo in range(8):
                            rows_v[cur, kk, pl.ds(dlo * L, L)] = (
                                rows_v[cur, kk, pl.ds(dlo * L, L)] * av)
                    return 0

                lax.fori_loop(0, BW // 4, scale, 0)
                puts[cur] = pltpu.async_copy(
                    rows_v.at[cur], shared_out.at[colc.at[jj]], sem_s,
                    add=True)
            for p in puts:
                if p is not None:
                    p.wait()
            return 0

        lax.fori_loop(0, chunks_real, p2_body, 0)
        plsc.subcore_barrier()

        pltpu.sync_copy(shared_out.at[pl.ds(s * stripe, stripe)],
                        out_hbm.at[pl.ds(s * stripe, stripe)])

    return sc_kernel


def kernel(x, edge_index, num_nodes, W, a):
    n, d_in = x.shape
    d_out = W.shape[0]
    e_cnt = edge_index.shape[1]
    lp = jnp.arange(n, dtype=jnp.int32)
    ei = jnp.concatenate(
        [edge_index.astype(jnp.int32), jnp.stack([lp, lp])], axis=1)
    e_total = e_cnt + n
    ep = -(-e_total // (16 * CH * BW)) * (16 * CH * BW)
    ei = jnp.pad(ei, ((0, 0), (0, ep - e_total)))
    rows64 = ep // BW
    row2d = ei[0].reshape(rows64, BW)
    col2d = ei[1].reshape(rows64, BW)

    a1 = a[:d_out, 0]
    a2 = a[d_out:, 0]
    apad = jnp.zeros((d_out, 128), jnp.float32)
    apad = apad.at[:, 0].set(a1).at[:, 1].set(a2)

    blk = 1000
    grid = n // blk
    h, sall = pl.pallas_call(
        _tc1_body,
        grid=(grid,),
        in_specs=[
            pl.BlockSpec((blk, d_in), lambda i: (i, 0)),
            pl.BlockSpec((d_out, d_in), lambda i: (0, 0)),
            pl.BlockSpec((d_out, 128), lambda i: (0, 0)),
        ],
        out_specs=[
            pl.BlockSpec((blk, 128), lambda i: (i, 0)),
            pl.BlockSpec((blk, 128), lambda i: (i, 0)),
        ],
        out_shape=[
            jax.ShapeDtypeStruct((n, 128), jnp.float32),
            jax.ShapeDtypeStruct((n, 128), jnp.float32),
        ],
    )(x, W, apad)

    n_pad = -(-n // (16 * 128)) * (16 * 128)
    pad_n = n_pad - n
    sdst = jnp.pad(sall[:, 0], (0, pad_n), constant_values=-3e38)
    ssrc = jnp.pad(sall[:, 1], (0, pad_n), constant_values=-3e38)

    sc = _make_sc_kernel(n, n_pad, rows64 // 16, e_total)
    partial, _ = sc(row2d, col2d, sdst, ssrc, h)
    return partial[:n]
